# baseline (speedup 1.0000x reference)
import functools

import jax
import jax.numpy as jnp
from jax import lax
from jax.experimental import pallas as pl
from jax.experimental.pallas import tpu as pltpu
from jax.experimental.pallas import tpu_sc as plsc

B = 16384
F = 12
D_MODEL = 768
FACTOR_DIM = 64
KPAD = 16

NC, NS = 2, 16
NW = NC * NS
ROWS_W = B // NW
GROUPS = ROWS_W // 16

BLK = 2048


def _sc_body(idx_hbm, val_hbm, out_hbm, idx_v, val_v, wmat_v):
    wid = lax.axis_index("s") * NC + lax.axis_index("c")
    base = wid * (ROWS_W * F)
    pltpu.sync_copy(idx_hbm.at[pl.ds(base, ROWS_W * F)], idx_v)
    pltpu.sync_copy(val_hbm.at[pl.ds(base, ROWS_W * F)], val_v)

    zeros = jnp.zeros((16,), jnp.float32)

    def zero_body(i, _):
        wmat_v[pl.ds(pl.multiple_of(i * 16, 16), 16)] = zeros
        return 0

    lax.fori_loop(0, ROWS_W * KPAD // 16, zero_body, 0)

    lanes = lax.iota(jnp.int32, 16)

    def group_body(g, _):
        lr = g * 16 + lanes
        for f in range(F):
            src = lr * F + f
            iv = plsc.load_gather(idx_v, [src])
            vv = plsc.load_gather(val_v, [src])
            vv = jnp.minimum(jnp.maximum(vv, 0.0), 1.0)
            plsc.addupdate_scatter(wmat_v, [lr * KPAD + iv], vv)
        return 0

    lax.fori_loop(0, GROUPS, group_body, 0)

    pltpu.sync_copy(wmat_v, out_hbm.at[pl.ds(wid * (ROWS_W * KPAD),
                                             ROWS_W * KPAD)])


_sc_wmat = pl.kernel(
    _sc_body,
    out_type=jax.ShapeDtypeStruct((B * KPAD,), jnp.float32),
    mesh=plsc.VectorSubcoreMesh(core_axis_name="c", subcore_axis_name="s"),
    compiler_params=pltpu.CompilerParams(needs_layout_passes=False),
    scratch_types=[
        pltpu.VMEM((ROWS_W * F,), jnp.int32),
        pltpu.VMEM((ROWS_W * F,), jnp.float32),
        pltpu.VMEM((ROWS_W * KPAD,), jnp.float32),
    ],
)


def _proj_body(w_ref, ftp_ref, wp_ref, b_ref, out_ref):
    m = jnp.dot(ftp_ref[...], wp_ref[...],
                preferred_element_type=jnp.float32)
    out_ref[...] = jnp.dot(w_ref[...], m,
                           preferred_element_type=jnp.float32) + b_ref[...]


def _project(wmat, ft_pad, W_proj, b2d):
    return pl.pallas_call(
        _proj_body,
        grid=(B // BLK,),
        in_specs=[
            pl.BlockSpec((BLK, KPAD), lambda i: (i, 0)),
            pl.BlockSpec((KPAD, FACTOR_DIM), lambda i: (0, 0)),
            pl.BlockSpec((FACTOR_DIM, D_MODEL), lambda i: (0, 0)),
            pl.BlockSpec((1, D_MODEL), lambda i: (0, 0)),
        ],
        out_specs=pl.BlockSpec((BLK, D_MODEL), lambda i: (i, 0)),
        out_shape=jax.ShapeDtypeStruct((B, D_MODEL), jnp.float32),
    )(wmat, ft_pad, W_proj, b2d)


@jax.jit
def _run(indices, values, factor_table, W_proj, b_proj):
    wmat = _sc_wmat(indices.reshape(-1), values.reshape(-1))
    wmat = wmat.reshape(B, KPAD)
    ft_pad = jnp.pad(factor_table, ((0, KPAD - F), (0, 0)))
    return _project(wmat, ft_pad, W_proj, b_proj.reshape(1, D_MODEL))


def kernel(indices, values, factor_table, W_proj, b_proj):
    return _run(indices, values, factor_table, W_proj, b_proj)


# speedup vs baseline: 1.0381x; 1.0381x over previous
import jax
import jax.numpy as jnp
from jax import lax
from jax.experimental import pallas as pl
from jax.experimental.pallas import tpu as pltpu
from jax.experimental.pallas import tpu_sc as plsc

B = 16384
F = 12
D_MODEL = 768
FACTOR_DIM = 64
KPAD = 16

NC, NS = 2, 16
NW = NC * NS
ROWS_W = B // NW
GROUPS = ROWS_W // 16
EPW = ROWS_W * F
IN_ROWS = B * F // 128
IN_ROWS_W = IN_ROWS // NW

BLK = 2048


def _sc_body(idx_hbm, val_hbm, out_hbm, idx_v, val_v, wmat_v):
    wid = lax.axis_index("s") * NC + lax.axis_index("c")
    pltpu.sync_copy(idx_hbm.at[pl.ds(wid * IN_ROWS_W, IN_ROWS_W), :], idx_v)
    pltpu.sync_copy(val_hbm.at[pl.ds(wid * IN_ROWS_W, IN_ROWS_W), :], val_v)

    zeros = jnp.zeros((16,), jnp.float32)

    def zero_body(i, _):
        wmat_v[i, :] = zeros
        return 0

    lax.fori_loop(0, ROWS_W, zero_body, 0)

    lanes = lax.iota(jnp.int32, 16)

    def group_body(g, _):
        lr = g * 16 + lanes
        for f in range(F):
            p = lr * F + f
            pr, pc = p // 128, p % 128
            iv = plsc.load_gather(idx_v, [pr, pc])
            vv = plsc.load_gather(val_v, [pr, pc])
            vv = jnp.minimum(jnp.maximum(vv, 0.0), 1.0)
            plsc.addupdate_scatter(wmat_v, [lr, iv], vv)
        return 0

    lax.fori_loop(0, GROUPS, group_body, 0)

    pltpu.sync_copy(wmat_v, out_hbm.at[pl.ds(wid * ROWS_W, ROWS_W), :])


_sc_wmat = pl.kernel(
    _sc_body,
    out_type=jax.ShapeDtypeStruct((B, KPAD), jnp.float32),
    mesh=plsc.VectorSubcoreMesh(core_axis_name="c", subcore_axis_name="s"),
    compiler_params=pltpu.CompilerParams(needs_layout_passes=False),
    scratch_types=[
        pltpu.VMEM((IN_ROWS_W, 128), jnp.int32),
        pltpu.VMEM((IN_ROWS_W, 128), jnp.float32),
        pltpu.VMEM((ROWS_W, KPAD), jnp.float32),
    ],
)


def _proj_body(w_ref, ftp_ref, wp_ref, b_ref, out_ref):
    m = jnp.dot(ftp_ref[...], wp_ref[...],
                preferred_element_type=jnp.float32)
    out_ref[...] = jnp.dot(w_ref[...], m,
                           preferred_element_type=jnp.float32) + b_ref[...]


def _project(wmat, ft_pad, W_proj, b2d):
    return pl.pallas_call(
        _proj_body,
        grid=(B // BLK,),
        in_specs=[
            pl.BlockSpec((BLK, KPAD), lambda i: (i, 0)),
            pl.BlockSpec((KPAD, FACTOR_DIM), lambda i: (0, 0)),
            pl.BlockSpec((FACTOR_DIM, D_MODEL), lambda i: (0, 0)),
            pl.BlockSpec((1, D_MODEL), lambda i: (0, 0)),
        ],
        out_specs=pl.BlockSpec((BLK, D_MODEL), lambda i: (i, 0)),
        out_shape=jax.ShapeDtypeStruct((B, D_MODEL), jnp.float32),
    )(wmat, ft_pad, W_proj, b2d)


@jax.jit
def _run(indices, values, factor_table, W_proj, b_proj):
    wmat = _sc_wmat(indices.reshape(IN_ROWS, 128),
                    values.reshape(IN_ROWS, 128))
    ft_pad = jnp.pad(factor_table, ((0, KPAD - F), (0, 0)))
    return _project(wmat, ft_pad, W_proj, b_proj.reshape(1, D_MODEL))


def kernel(indices, values, factor_table, W_proj, b_proj):
    return _run(indices, values, factor_table, W_proj, b_proj)
